# baseline (speedup 1.0000x reference)
import jax
import jax.numpy as jnp
from jax.experimental import pallas as pl

_CB = 8


def _fused_kernel(x_ref, u_ref, tv_ref, w_ref, b_ref, out_ref):
    cb, m = u_ref.shape
    _, m2, d2 = tv_ref.shape
    d = d2 // 2
    emb_all = jnp.dot(x_ref[...], w_ref[...], preferred_element_type=jnp.float32)
    emb_all = emb_all + b_ref[...]
    u = u_ref[...]
    col = jax.lax.broadcasted_iota(jnp.int32, (cb, m), 1)
    rows2 = jax.lax.broadcasted_iota(jnp.int32, (m2, d2), 0)
    halves2 = jax.lax.broadcasted_iota(jnp.int32, (m2, d2), 1) // d
    for r in range(cb):
        ur = u[r : r + 1, :]
        umin = jnp.min(ur)
        midx = jnp.min(jnp.where(ur == umin, col[r : r + 1, :], m))
        emb = emb_all[r : r + 1, :]
        emb_t = jnp.concatenate([emb, emb], axis=1)
        mask = (rows2 == midx // 2) & (halves2 == midx % 2)
        out_ref[r] = jnp.where(mask, emb_t, tv_ref[r])


def kernel(x, token_values, token_usages, W_embed, b_embed):
    B, M, D = token_values.shape
    tv2 = token_values.reshape(B, M // 2, 2 * D)
    grid = (B // _CB,)
    out = pl.pallas_call(
        _fused_kernel,
        grid=grid,
        in_specs=[
            pl.BlockSpec((_CB, D), lambda i: (i, 0)),
            pl.BlockSpec((_CB, M), lambda i: (i, 0)),
            pl.BlockSpec((_CB, M // 2, 2 * D), lambda i: (i, 0, 0)),
            pl.BlockSpec((D, D), lambda i: (0, 0)),
            pl.BlockSpec((1, D), lambda i: (0, 0)),
        ],
        out_specs=pl.BlockSpec((_CB, M // 2, 2 * D), lambda i: (i, 0, 0)),
        out_shape=jax.ShapeDtypeStruct((B, M // 2, 2 * D), jnp.float32),
    )(x, token_usages, tv2, W_embed, b_embed.reshape(1, D))
    return out.reshape(B, M, D)


# speedup vs baseline: 1.0962x; 1.0962x over previous
import jax
import jax.numpy as jnp
from jax.experimental import pallas as pl

_CB = 8


def _fused_kernel(x_ref, u_ref, tv_ref, w_ref, b_ref, out_ref):
    cb, m = u_ref.shape
    emb_all = jnp.dot(x_ref[...], w_ref[...], preferred_element_type=jnp.float32)
    emb_all = emb_all + b_ref[...]
    u = u_ref[...]
    col = jax.lax.broadcasted_iota(jnp.int32, (cb, m), 1)
    umin = jnp.min(u, axis=1, keepdims=True)
    midx = jnp.min(jnp.where(u == umin, col, m), axis=1, keepdims=True)
    out_ref[...] = tv_ref[...]
    for r in range(cb):
        s = midx[r, 0]
        out_ref[r, pl.ds(s, 1), :] = emb_all[r : r + 1, :]


def kernel(x, token_values, token_usages, W_embed, b_embed):
    B, M, D = token_values.shape
    grid = (B // _CB,)
    return pl.pallas_call(
        _fused_kernel,
        grid=grid,
        in_specs=[
            pl.BlockSpec((_CB, D), lambda i: (i, 0)),
            pl.BlockSpec((_CB, M), lambda i: (i, 0)),
            pl.BlockSpec((_CB, M, D), lambda i: (i, 0, 0)),
            pl.BlockSpec((D, D), lambda i: (0, 0)),
            pl.BlockSpec((1, D), lambda i: (0, 0)),
        ],
        out_specs=pl.BlockSpec((_CB, M, D), lambda i: (i, 0, 0)),
        out_shape=jax.ShapeDtypeStruct((B, M, D), jnp.float32),
    )(x, token_usages, token_values, W_embed, b_embed.reshape(1, D))
